# Initial kernel scaffold; baseline (speedup 1.0000x reference)
import jax
import jax.numpy as jnp
from jax.experimental import pallas as pl


def kernel(z, edge_index):
    raise NotImplementedError("write your pallas kernel here")


# speedup vs baseline: 2.9552x; 2.9552x over previous
import functools

import jax
import jax.numpy as jnp
from jax import lax
from jax.experimental import pallas as pl
from jax.experimental.pallas import tpu as pltpu
from jax.experimental.pallas import tpu_sc as plsc

D = 128
E = 320000
NW = 32
EPW = E // NW
C = 80
NCHUNK = EPW // C


def _decoder_kernel(z_hbm, srci_hbm, dsti_hbm, out_hbm,
                    srci_v, dsti_v, src_rows, dst_rows, out_v, sem):
    wid = lax.axis_index("s") * 2 + lax.axis_index("c")

    pltpu.sync_copy(srci_hbm.at[wid], srci_v)
    pltpu.sync_copy(dsti_hbm.at[wid], dsti_v)

    lane = lax.iota(jnp.int32, 16)

    def chunk_body(ci, carry):
        cp_s = pltpu.async_copy(z_hbm.at[srci_v.at[ci]], src_rows, sem)
        cp_d = pltpu.async_copy(z_hbm.at[dsti_v.at[ci]], dst_rows, sem)
        cp_s.wait()
        cp_d.wait()

        for g in range(C // 16):
            vec = jnp.zeros((16,), jnp.float32)
            for l in range(16):
                e = g * 16 + l
                acc = src_rows[e, pl.ds(0, 16)] * dst_rows[e, pl.ds(0, 16)]
                for j in range(1, D // 16):
                    acc = acc + (src_rows[e, pl.ds(j * 16, 16)]
                                 * dst_rows[e, pl.ds(j * 16, 16)])
                vec = jnp.where(lane == l, jnp.sum(acc), vec)
            out_v[pl.ds(ci * C + g * 16, 16)] = vec
        return carry

    lax.fori_loop(0, NCHUNK, chunk_body, 0)

    pltpu.sync_copy(out_v, out_hbm.at[wid])


@jax.jit
def kernel(z, edge_index):
    ei = edge_index.astype(jnp.int32)
    srci = ei[0].reshape(NW, NCHUNK, C)
    dsti = ei[1].reshape(NW, NCHUNK, C)

    mesh = plsc.VectorSubcoreMesh(core_axis_name="c", subcore_axis_name="s")
    run = pl.kernel(
        _decoder_kernel,
        mesh=mesh,
        compiler_params=pltpu.CompilerParams(
            needs_layout_passes=False,
            use_tc_tiling_on_sc=False,
        ),
        out_type=jax.ShapeDtypeStruct((NW, EPW), jnp.float32),
        scratch_types=[
            pltpu.VMEM((NCHUNK, C), jnp.int32),
            pltpu.VMEM((NCHUNK, C), jnp.int32),
            pltpu.VMEM((C, D), jnp.float32),
            pltpu.VMEM((C, D), jnp.float32),
            pltpu.VMEM((EPW,), jnp.float32),
            pltpu.SemaphoreType.DMA,
        ],
    )
    out = run(z, srci, dsti)
    return out.reshape(E)
